# baseline (device time: 139340 ns/iter reference)
import jax
import jax.numpy as jnp
from jax import lax
from jax.experimental import pallas as pl
from jax.experimental.pallas import tpu as pltpu

N_DEV = 8
BF16 = jnp.bfloat16
EPS = 1e-5


def _barrier_all(my):
    barrier_sem = pltpu.get_barrier_semaphore()
    for k in range(1, N_DEV):
        pl.semaphore_signal(
            barrier_sem, inc=1,
            device_id=(lax.rem(my + k, N_DEV),),
            device_id_type=pl.DeviceIdType.MESH,
        )
    pl.semaphore_wait(barrier_sem, N_DEV - 1)


def _a2a_allreduce(chunk_fn, my, ar_ref, bufs, sems):
    send_buf, comm_ref, ag_stage, ag_comm = bufs
    rs_send, rs_recv, ag_send, ag_recv = sems
    chunk = ar_ref.shape[0] // N_DEV

    rs = []
    for k in range(1, N_DEV):
        target = lax.rem(my + k, N_DEV)
        send_buf[k - 1] = chunk_fn(target * chunk).astype(send_buf.dtype)
        rdma = pltpu.make_async_remote_copy(
            src_ref=send_buf.at[k - 1],
            dst_ref=comm_ref.at[k - 1],
            send_sem=rs_send.at[k - 1],
            recv_sem=rs_recv.at[k - 1],
            device_id=(target,),
            device_id_type=pl.DeviceIdType.MESH,
        )
        rdma.start()
        rs.append(rdma)

    ar_ref[pl.ds(my * chunk, chunk), :] = chunk_fn(my * chunk).astype(ar_ref.dtype)
    for k in range(1, N_DEV):
        rs[k - 1].wait_recv()
        ar_ref[pl.ds(my * chunk, chunk), :] += comm_ref[k - 1].astype(ar_ref.dtype)

    ag_stage[...] = ar_ref[pl.ds(my * chunk, chunk), :].astype(ag_stage.dtype)
    ag = []
    for k in range(1, N_DEV):
        target = lax.rem(my + k, N_DEV)
        rdma = pltpu.make_async_remote_copy(
            src_ref=ag_stage,
            dst_ref=ag_comm.at[k - 1],
            send_sem=ag_send.at[k - 1],
            recv_sem=ag_recv.at[k - 1],
            device_id=(target,),
            device_id_type=pl.DeviceIdType.MESH,
        )
        rdma.start()
        ag.append(rdma)

    for k in range(1, N_DEV):
        ag[k - 1].wait_recv()
        owner = lax.rem(my - k + N_DEV, N_DEV)
        ar_ref[pl.ds(owner * chunk, chunk), :] = ag_comm[k - 1].astype(ar_ref.dtype)
    for k in range(1, N_DEV):
        rs[k - 1].wait_send()
        ag[k - 1].wait_send()


FP8 = jnp.float8_e4m3fn


def _sem_scratch(chunk, cols):
    return [
        pltpu.VMEM((N_DEV - 1, chunk, cols), FP8),
        pltpu.VMEM((N_DEV - 1, chunk, cols), FP8),
        pltpu.VMEM((chunk, cols), FP8),
        pltpu.VMEM((N_DEV - 1, chunk, cols), FP8),
        pltpu.SemaphoreType.DMA((N_DEV - 1,)),
        pltpu.SemaphoreType.DMA((N_DEV - 1,)),
        pltpu.SemaphoreType.DMA((N_DEV - 1,)),
        pltpu.SemaphoreType.DMA((N_DEV - 1,)),
    ]


def _attn_out_body(q_ref, k_ref, v_ref, wo_ref, out_ref,
                   send_buf, comm_ref, ag_stage, ag_comm, *sems):
    my = lax.axis_index("i")
    _barrier_all(my)
    rows, D = out_ref.shape
    S = k_ref.shape[2]
    Dh = k_ref.shape[1]
    H = k_ref.shape[0] // (rows // S)
    chunk = rows // N_DEV

    def chunk_fn(start):
        b = start // S

        acc = jnp.zeros((chunk, D), jnp.float32)
        for h in range(H):
            q = q_ref[pl.ds(start, chunk), pl.ds(h * Dh, Dh)]
            kt = k_ref[pl.ds(b * H + h, 1), :, :][0]
            v = v_ref[pl.ds(b * S, S), pl.ds(h * Dh, Dh)]
            s = jnp.dot(q, kt, preferred_element_type=jnp.float32)
            e = jnp.exp(s.astype(BF16))
            r = 1.0 / jnp.sum(e, axis=-1, keepdims=True, dtype=jnp.float32)
            pv = jnp.dot(e, v, preferred_element_type=jnp.float32)
            pv = (pv * r).astype(BF16)
            wo_h = wo_ref[pl.ds(h * Dh, Dh), :]
            acc = acc + jnp.dot(pv, wo_h, preferred_element_type=jnp.float32)
        return acc.astype(BF16)

    _a2a_allreduce(chunk_fn, my, out_ref,
                   (send_buf, comm_ref, ag_stage, ag_comm), sems)


def _attn_out_allreduce(q, k, v, wo):
    rows = q.shape[0]
    cols = wo.shape[1]
    return pl.pallas_call(
        _attn_out_body,
        out_shape=jax.ShapeDtypeStruct((rows, cols), BF16),
        in_specs=[pl.BlockSpec(memory_space=pltpu.VMEM)] * 4,
        out_specs=pl.BlockSpec(memory_space=pltpu.VMEM),
        scratch_shapes=_sem_scratch(rows // N_DEV, cols),
        compiler_params=pltpu.CompilerParams(collective_id=0),
    )(q, k, v, wo)


def _block2_body(x0_ref, attn_ref, mod_ref, w1_ref, w2_ref,
                 out_ref, x1_ref, ff_ref,
                 send_buf, comm_ref, ag_stage, ag_comm, *sems):
    rows, D = out_ref.shape
    S = rows // 2
    chunk = rows // N_DEV
    my = lax.axis_index("i")
    _barrier_all(my)

    def chunk_fn(start):
        b = start // S
        ga = mod_ref[pl.ds(b, 1), pl.ds(2 * D, D)]
        xc = (
            x0_ref[pl.ds(start, chunk), :]
            + ga * attn_ref[pl.ds(start, chunk), :].astype(jnp.float32)
        )
        x1_ref[pl.ds(start, chunk), :] = xc
        sm = mod_ref[pl.ds(b, 1), pl.ds(3 * D, D)]
        shm = mod_ref[pl.ds(b, 1), pl.ds(4 * D, D)]
        m = jnp.mean(xc, axis=-1, keepdims=True)
        v = jnp.mean(jnp.square(xc - m), axis=-1, keepdims=True)
        xn = ((xc - m) * lax.rsqrt(v + EPS) * (1.0 + sm) + shm).astype(BF16)
        h = jnp.dot(xn, w1_ref[...], preferred_element_type=jnp.float32)
        h = (h * jax.nn.sigmoid(h)).astype(BF16)
        return jnp.dot(h, w2_ref[...], preferred_element_type=jnp.float32).astype(BF16)

    _a2a_allreduce(chunk_fn, my, ff_ref,
                   (send_buf, comm_ref, ag_stage, ag_comm), sems)

    for b in range(2):
        gm = mod_ref[b, pl.ds(5 * D, D)][None, :]
        out_ref[pl.ds(b * S, S), :] = (
            x1_ref[pl.ds(b * S, S), :]
            + gm * ff_ref[pl.ds(b * S, S), :].astype(jnp.float32)
        )


def _block2(x0, attn_sum, mod, w1, w2):
    rows, D = x0.shape
    return pl.pallas_call(
        _block2_body,
        out_shape=jax.ShapeDtypeStruct((rows, D), jnp.float32),
        in_specs=[pl.BlockSpec(memory_space=pltpu.VMEM)] * 5,
        out_specs=pl.BlockSpec(memory_space=pltpu.VMEM),
        scratch_shapes=[
            pltpu.VMEM((rows, D), jnp.float32),
            pltpu.VMEM((rows, D), BF16),
        ] + _sem_scratch(rows // N_DEV, D),
        compiler_params=pltpu.CompilerParams(collective_id=1),
    )(x0, attn_sum, mod, w1, w2)


def kernel(x, Wq, Wk, Wv, Wo, t_emb, W_mod, W_ff1, W_ff2):
    B, S, D = x.shape
    Dh = 128
    H = Wq.shape[1] // Dh
    scale = 0.08838834764831843

    mod = t_emb @ W_mod
    sa, sha = mod[:, :D], mod[:, D:2 * D]

    x0 = x.reshape(B * S, D)
    m = jnp.mean(x, axis=-1, keepdims=True)
    v = jnp.var(x, axis=-1, keepdims=True)
    xm = ((x - m) * lax.rsqrt(v + EPS) * (1.0 + sa[:, None, :])
          + sha[:, None, :]).astype(BF16)

    Wqkv = jnp.concatenate(
        [(Wq * scale).astype(BF16), Wk.astype(BF16), Wv.astype(BF16)], axis=1
    )
    qkv = (xm @ Wqkv).reshape(B * S, 3 * H * Dh)
    HD = H * Dh
    Q = qkv[:, :HD]
    V = qkv[:, 2 * HD:]
    K = qkv[:, HD:2 * HD].reshape(B, S, H, Dh)
    K = K.transpose(0, 2, 3, 1).reshape(B * H, Dh, S)

    attn_sum = _attn_out_allreduce(Q, K, V, Wo.astype(BF16))

    out = _block2(x0, attn_sum, mod, W_ff1.astype(BF16), W_ff2.astype(BF16))
    return out.reshape(B, S, D)


# device time: 123080 ns/iter; 1.1321x vs baseline; 1.1321x over previous
import jax
import jax.numpy as jnp
from jax import lax
from jax.experimental import pallas as pl
from jax.experimental.pallas import tpu as pltpu

N_DEV = 8
BF16 = jnp.bfloat16
FP8 = jnp.float8_e4m3fn
EPS = 1e-5


def _barrier_all(my):
    barrier_sem = pltpu.get_barrier_semaphore()
    for k in range(1, N_DEV):
        pl.semaphore_signal(
            barrier_sem, inc=1,
            device_id=(lax.rem(my + k, N_DEV),),
            device_id_type=pl.DeviceIdType.MESH,
        )
    pl.semaphore_wait(barrier_sem, N_DEV - 1)


def _fused_block_body(q_ref, k_ref, v_ref, wo_ref, x0_ref, mod_ref,
                      w1_ref, w2_ref, out_ref,
                      attn_buf, x1_ref, ff_ref,
                      rs1_send_buf, rs1_comm, ag1_stage, ag1_comm,
                      rs2_send_buf, rs2_comm, ag2_stage, ag2_comm,
                      rs1_send, rs1_recv, ag1_send, ag1_recv,
                      rs2_send, rs2_recv, ag2_send, ag2_recv):
    rows, D = out_ref.shape
    S = k_ref.shape[2]
    Dh = k_ref.shape[1]
    H = k_ref.shape[0] // (rows // S)
    chunk = rows // N_DEV
    my = lax.axis_index("i")
    _barrier_all(my)

    def attn_chunk(start):
        b = start // S
        acc = jnp.zeros((chunk, D), jnp.float32)
        for h in range(H):
            q = q_ref[pl.ds(start, chunk), pl.ds(h * Dh, Dh)]
            kt = k_ref[pl.ds(b * H + h, 1), :, :][0]
            v = v_ref[pl.ds(b * S, S), pl.ds(h * Dh, Dh)]
            e = jnp.exp(jnp.dot(q, kt, preferred_element_type=jnp.float32))
            r = 1.0 / jnp.sum(e, axis=-1, keepdims=True)
            pv = jnp.dot(e.astype(BF16), v, preferred_element_type=jnp.float32)
            pv = (pv * r).astype(BF16)
            wo_h = wo_ref[pl.ds(h * Dh, Dh), :]
            acc = acc + jnp.dot(pv, wo_h, preferred_element_type=jnp.float32)
        return acc

    def ffn_chunk(start):
        b = start // S
        ga = mod_ref[pl.ds(b, 1), pl.ds(2 * D, D)]
        xc = (
            x0_ref[pl.ds(start, chunk), :].astype(jnp.float32)
            + ga * attn_buf[pl.ds(start, chunk), :].astype(jnp.float32)
        )
        x1_ref[pl.ds(start, chunk), :] = xc.astype(x1_ref.dtype)
        sm = mod_ref[pl.ds(b, 1), pl.ds(3 * D, D)]
        shm = mod_ref[pl.ds(b, 1), pl.ds(4 * D, D)]
        m = jnp.mean(xc, axis=-1, keepdims=True)
        var = jnp.mean(jnp.square(xc - m), axis=-1, keepdims=True)
        xn = ((xc - m) * lax.rsqrt(var + EPS) * (1.0 + sm) + shm).astype(BF16)
        h = jnp.dot(xn, w1_ref[...], preferred_element_type=jnp.float32)
        h = (h * jax.nn.sigmoid(h)).astype(BF16)
        return jnp.dot(h, w2_ref[...], preferred_element_type=jnp.float32)

    rs1 = []
    for k in range(1, N_DEV):
        target = lax.rem(my + k, N_DEV)
        rs1_send_buf[k - 1] = attn_chunk(target * chunk).astype(FP8)
        rdma = pltpu.make_async_remote_copy(
            src_ref=rs1_send_buf.at[k - 1],
            dst_ref=rs1_comm.at[k - 1],
            send_sem=rs1_send.at[k - 1],
            recv_sem=rs1_recv.at[k - 1],
            device_id=(target,),
            device_id_type=pl.DeviceIdType.MESH,
        )
        rdma.start()
        rs1.append(rdma)

    attn_buf[pl.ds(my * chunk, chunk), :] = attn_chunk(my * chunk).astype(BF16)
    for k in range(1, N_DEV):
        rs1[k - 1].wait_recv()
        attn_buf[pl.ds(my * chunk, chunk), :] += rs1_comm[k - 1].astype(BF16)

    ag1_stage[...] = attn_buf[pl.ds(my * chunk, chunk), :].astype(FP8)
    ag1 = []
    for k in range(1, N_DEV):
        target = lax.rem(my + k, N_DEV)
        rdma = pltpu.make_async_remote_copy(
            src_ref=ag1_stage,
            dst_ref=ag1_comm.at[k - 1],
            send_sem=ag1_send.at[k - 1],
            recv_sem=ag1_recv.at[k - 1],
            device_id=(target,),
            device_id_type=pl.DeviceIdType.MESH,
        )
        rdma.start()
        ag1.append(rdma)

    ff_ref[pl.ds(my * chunk, chunk), :] = ffn_chunk(my * chunk).astype(BF16)

    rs2 = []
    for k in range(1, N_DEV):
        ag1[k - 1].wait_recv()
        owner = lax.rem(my - k + N_DEV, N_DEV)
        attn_buf[pl.ds(owner * chunk, chunk), :] = ag1_comm[k - 1].astype(BF16)
        slot = N_DEV - 1 - k
        rs2_send_buf[slot] = ffn_chunk(owner * chunk).astype(FP8)
        rdma = pltpu.make_async_remote_copy(
            src_ref=rs2_send_buf.at[slot],
            dst_ref=rs2_comm.at[slot],
            send_sem=rs2_send.at[slot],
            recv_sem=rs2_recv.at[slot],
            device_id=(owner,),
            device_id_type=pl.DeviceIdType.MESH,
        )
        rdma.start()
        rs2.append(rdma)

    for j in range(N_DEV - 1):
        rs2[j].wait_recv()
    for j in range(N_DEV - 1):
        ff_ref[pl.ds(my * chunk, chunk), :] += rs2_comm[j].astype(BF16)

    ag2_stage[...] = ff_ref[pl.ds(my * chunk, chunk), :].astype(FP8)
    ag2 = []
    for k in range(1, N_DEV):
        target = lax.rem(my + k, N_DEV)
        rdma = pltpu.make_async_remote_copy(
            src_ref=ag2_stage,
            dst_ref=ag2_comm.at[k - 1],
            send_sem=ag2_send.at[k - 1],
            recv_sem=ag2_recv.at[k - 1],
            device_id=(target,),
            device_id_type=pl.DeviceIdType.MESH,
        )
        rdma.start()
        ag2.append(rdma)
    for k in range(1, N_DEV):
        ag2[k - 1].wait_recv()
        owner = lax.rem(my - k + N_DEV, N_DEV)
        ff_ref[pl.ds(owner * chunk, chunk), :] = ag2_comm[k - 1].astype(BF16)

    for b in range(2):
        gm = mod_ref[b, pl.ds(5 * D, D)][None, :]
        out_ref[pl.ds(b * S, S), :] = (
            x1_ref[pl.ds(b * S, S), :].astype(jnp.float32)
            + gm * ff_ref[pl.ds(b * S, S), :].astype(jnp.float32)
        )

    for k in range(N_DEV - 1):
        rs1[k].wait_send()
        ag1[k].wait_send()
        rs2[k].wait_send()
        ag2[k].wait_send()


def _fused_block(q, k, v, wo, x0, mod, w1, w2):
    rows, D = x0.shape
    chunk = rows // N_DEV

    def comm_bufs():
        return [
            pltpu.VMEM((N_DEV - 1, chunk, D), FP8),
            pltpu.VMEM((N_DEV - 1, chunk, D), FP8),
            pltpu.VMEM((chunk, D), FP8),
            pltpu.VMEM((N_DEV - 1, chunk, D), FP8),
        ]

    return pl.pallas_call(
        _fused_block_body,
        out_shape=jax.ShapeDtypeStruct((rows, D), jnp.float32),
        in_specs=[pl.BlockSpec(memory_space=pltpu.VMEM)] * 8,
        out_specs=pl.BlockSpec(memory_space=pltpu.VMEM),
        scratch_shapes=[
            pltpu.VMEM((rows, D), BF16),
            pltpu.VMEM((rows, D), BF16),
            pltpu.VMEM((rows, D), BF16),
        ] + comm_bufs() + comm_bufs() + [
            pltpu.SemaphoreType.DMA((N_DEV - 1,)) for _ in range(8)
        ],
        compiler_params=pltpu.CompilerParams(collective_id=0),
    )(q, k, v, wo, x0, mod, w1, w2)


def kernel(x, Wq, Wk, Wv, Wo, t_emb, W_mod, W_ff1, W_ff2):
    B, S, D = x.shape
    Dh = 128
    H = Wq.shape[1] // Dh
    scale = 0.08838834764831843

    mod = t_emb @ W_mod
    sa, sha = mod[:, :D], mod[:, D:2 * D]

    m = jnp.mean(x, axis=-1, keepdims=True)
    v = jnp.var(x, axis=-1, keepdims=True)
    xm = ((x - m) * lax.rsqrt(v + EPS) * (1.0 + sa[:, None, :])
          + sha[:, None, :]).astype(BF16)

    Q = (xm @ (Wq * scale).astype(BF16)).reshape(B * S, H * Dh)
    V = (xm @ Wv.astype(BF16)).reshape(B * S, H * Dh)
    K = (xm @ Wk.astype(BF16)).reshape(B, S, H, Dh)
    K = K.transpose(0, 2, 3, 1).reshape(B * H, Dh, S)

    out = _fused_block(
        Q, K, V, Wo.astype(BF16),
        x.reshape(B * S, D).astype(BF16), mod,
        W_ff1.astype(BF16), W_ff2.astype(BF16),
    )
    return out.reshape(B, S, D)


# device time: 119271 ns/iter; 1.1683x vs baseline; 1.0319x over previous
import jax
import jax.numpy as jnp
from jax import lax
from jax.experimental import pallas as pl
from jax.experimental.pallas import tpu as pltpu

N_DEV = 8
BF16 = jnp.bfloat16
FP8 = jnp.float8_e4m3fn
EPS = 1e-5


def _barrier_all(my):
    barrier_sem = pltpu.get_barrier_semaphore()
    for k in range(1, N_DEV):
        pl.semaphore_signal(
            barrier_sem, inc=1,
            device_id=(lax.rem(my + k, N_DEV),),
            device_id_type=pl.DeviceIdType.MESH,
        )
    pl.semaphore_wait(barrier_sem, N_DEV - 1)


def _fused_block_body(q_ref, k_ref, v_ref, wo_ref, x0_ref, mod_ref,
                      w1_ref, w2_ref, out_ref,
                      attn_buf, x1_ref, ff_ref,
                      rs1_send_buf, rs1_comm, ag1_stage, ag1_comm,
                      rs2_send_buf, rs2_comm, ag2_stage, ag2_comm,
                      rs1_send, rs1_recv, ag1_send, ag1_recv,
                      rs2_send, rs2_recv, ag2_send, ag2_recv):
    rows, D = out_ref.shape
    S = k_ref.shape[2]
    Dh = k_ref.shape[1]
    H = k_ref.shape[0] // (rows // S)
    chunk = rows // N_DEV
    my = lax.axis_index("i")
    _barrier_all(my)

    def attn_chunk(start):
        b = start // S
        acc = jnp.zeros((chunk, D), jnp.float32)
        for h in range(H):
            q = q_ref[pl.ds(start, chunk), pl.ds(h * Dh, Dh)]
            kt = k_ref[pl.ds(b * H + h, 1), :, :][0]
            v = v_ref[pl.ds(b * S, S), pl.ds(h * Dh, Dh)]
            e = jnp.exp(jnp.dot(q, kt, preferred_element_type=jnp.float32))
            r = 1.0 / jnp.sum(e, axis=-1, keepdims=True)
            pv = jnp.dot(e.astype(BF16), v, preferred_element_type=jnp.float32)
            pv = (pv * r).astype(BF16)
            wo_h = wo_ref[pl.ds(h * Dh, Dh), :]
            acc = acc + jnp.dot(pv, wo_h, preferred_element_type=jnp.float32)
        return acc

    def ffn_chunk(start):
        b = start // S
        ga = mod_ref[pl.ds(b, 1), pl.ds(2 * D, D)]
        xc = (
            x0_ref[pl.ds(start, chunk), :].astype(jnp.float32)
            + ga * attn_buf[pl.ds(start, chunk), :].astype(jnp.float32)
        )
        x1_ref[pl.ds(start, chunk), :] = xc.astype(x1_ref.dtype)
        sm = mod_ref[pl.ds(b, 1), pl.ds(3 * D, D)]
        shm = mod_ref[pl.ds(b, 1), pl.ds(4 * D, D)]
        m = jnp.mean(xc, axis=-1, keepdims=True)
        var = jnp.mean(jnp.square(xc - m), axis=-1, keepdims=True)
        xn = ((xc - m) * lax.rsqrt(var + EPS) * (1.0 + sm) + shm).astype(BF16)
        h = jnp.dot(xn, w1_ref[...], preferred_element_type=jnp.float32)
        h = (h * jax.nn.sigmoid(h)).astype(BF16)
        return jnp.dot(h, w2_ref[...], preferred_element_type=jnp.float32)

    rs1 = []
    for k in range(1, N_DEV):
        target = lax.rem(my + k, N_DEV)
        rs1_send_buf[k - 1] = attn_chunk(target * chunk).astype(FP8)
        rdma = pltpu.make_async_remote_copy(
            src_ref=rs1_send_buf.at[k - 1],
            dst_ref=rs1_comm.at[k - 1],
            send_sem=rs1_send.at[k - 1],
            recv_sem=rs1_recv.at[k - 1],
            device_id=(target,),
            device_id_type=pl.DeviceIdType.MESH,
        )
        rdma.start()
        rs1.append(rdma)

    attn_buf[pl.ds(my * chunk, chunk), :] = attn_chunk(my * chunk).astype(BF16)
    for k in range(1, N_DEV):
        rs1[k - 1].wait_recv()
        attn_buf[pl.ds(my * chunk, chunk), :] += rs1_comm[k - 1].astype(BF16)

    ag1_stage[...] = attn_buf[pl.ds(my * chunk, chunk), :].astype(FP8)
    ag1 = []
    for k in range(1, N_DEV):
        target = lax.rem(my + k, N_DEV)
        rdma = pltpu.make_async_remote_copy(
            src_ref=ag1_stage,
            dst_ref=ag1_comm.at[k - 1],
            send_sem=ag1_send.at[k - 1],
            recv_sem=ag1_recv.at[k - 1],
            device_id=(target,),
            device_id_type=pl.DeviceIdType.MESH,
        )
        rdma.start()
        ag1.append(rdma)

    ff_ref[pl.ds(my * chunk, chunk), :] = ffn_chunk(my * chunk).astype(BF16)

    rs2 = []
    for k in range(1, N_DEV):
        ag1[k - 1].wait_recv()
        owner = lax.rem(my - k + N_DEV, N_DEV)
        attn_buf[pl.ds(owner * chunk, chunk), :] = ag1_comm[k - 1].astype(BF16)
        slot = N_DEV - 1 - k
        rs2_send_buf[slot] = ffn_chunk(owner * chunk).astype(FP8)
        rdma = pltpu.make_async_remote_copy(
            src_ref=rs2_send_buf.at[slot],
            dst_ref=rs2_comm.at[slot],
            send_sem=rs2_send.at[slot],
            recv_sem=rs2_recv.at[slot],
            device_id=(owner,),
            device_id_type=pl.DeviceIdType.MESH,
        )
        rdma.start()
        rs2.append(rdma)

    for j in range(N_DEV - 1):
        rs2[j].wait_recv()
    for j in range(N_DEV - 1):
        ff_ref[pl.ds(my * chunk, chunk), :] += rs2_comm[j].astype(BF16)

    ag2_stage[...] = ff_ref[pl.ds(my * chunk, chunk), :].astype(FP8)
    ag2 = []
    for k in range(1, N_DEV):
        target = lax.rem(my + k, N_DEV)
        rdma = pltpu.make_async_remote_copy(
            src_ref=ag2_stage,
            dst_ref=ag2_comm.at[k - 1],
            send_sem=ag2_send.at[k - 1],
            recv_sem=ag2_recv.at[k - 1],
            device_id=(target,),
            device_id_type=pl.DeviceIdType.MESH,
        )
        rdma.start()
        ag2.append(rdma)
    for k in range(1, N_DEV):
        ag2[k - 1].wait_recv()
        owner = lax.rem(my - k + N_DEV, N_DEV)
        ff_ref[pl.ds(owner * chunk, chunk), :] = ag2_comm[k - 1].astype(BF16)

    for b in range(2):
        gm = mod_ref[b, pl.ds(5 * D, D)][None, :]
        out_ref[pl.ds(b * S, S), :] = (
            x1_ref[pl.ds(b * S, S), :].astype(jnp.float32)
            + gm * ff_ref[pl.ds(b * S, S), :].astype(jnp.float32)
        ).astype(BF16)

    for k in range(N_DEV - 1):
        rs1[k].wait_send()
        ag1[k].wait_send()
        rs2[k].wait_send()
        ag2[k].wait_send()


def _fused_block(q, k, v, wo, x0, mod, w1, w2):
    rows, D = x0.shape
    chunk = rows // N_DEV

    def comm_bufs():
        return [
            pltpu.VMEM((N_DEV - 1, chunk, D), FP8),
            pltpu.VMEM((N_DEV - 1, chunk, D), FP8),
            pltpu.VMEM((chunk, D), FP8),
            pltpu.VMEM((N_DEV - 1, chunk, D), FP8),
        ]

    return pl.pallas_call(
        _fused_block_body,
        out_shape=jax.ShapeDtypeStruct((rows, D), BF16),
        in_specs=[pl.BlockSpec(memory_space=pltpu.VMEM)] * 8,
        out_specs=pl.BlockSpec(memory_space=pltpu.VMEM),
        scratch_shapes=[
            pltpu.VMEM((rows, D), BF16),
            pltpu.VMEM((rows, D), BF16),
            pltpu.VMEM((rows, D), BF16),
        ] + comm_bufs() + comm_bufs() + [
            pltpu.SemaphoreType.DMA((N_DEV - 1,)) for _ in range(8)
        ],
        compiler_params=pltpu.CompilerParams(collective_id=0),
    )(q, k, v, wo, x0, mod, w1, w2)


def kernel(x, Wq, Wk, Wv, Wo, t_emb, W_mod, W_ff1, W_ff2):
    B, S, D = x.shape
    Dh = 128
    H = Wq.shape[1] // Dh
    scale = 0.08838834764831843

    mod = t_emb @ W_mod
    sa, sha = mod[:, :D], mod[:, D:2 * D]

    m = jnp.mean(x, axis=-1, keepdims=True)
    v = jnp.var(x, axis=-1, keepdims=True)
    xm = ((x - m) * lax.rsqrt(v + EPS) * (1.0 + sa[:, None, :])
          + sha[:, None, :]).astype(BF16)

    Q = (xm @ (Wq * scale).astype(BF16)).reshape(B * S, H * Dh)
    V = (xm @ Wv.astype(BF16)).reshape(B * S, H * Dh)
    K = (xm @ Wk.astype(BF16)).reshape(B, S, H, Dh)
    K = K.transpose(0, 2, 3, 1).reshape(B * H, Dh, S)

    out = _fused_block(
        Q, K, V, Wo.astype(BF16),
        x.reshape(B * S, D).astype(BF16), mod,
        W_ff1.astype(BF16), W_ff2.astype(BF16),
    )
    return out.reshape(B, S, D)
